# baseline (speedup 1.0000x reference)
import functools
import math

import jax
import jax.numpy as jnp
from jax import lax
from jax.experimental import pallas as pl
from jax.experimental.pallas import tpu as pltpu
from jax.experimental.pallas import tpu_sc as plsc

NMOD = 3
R = 16
NFF = 128
B = 4096

_NC = 2
_NS = 16
_NW = _NC * _NS
_BPW = B // _NW


def _gather_body(i0, i1, i2, u0, u1, u2, out, idx0, idx1, idx2, r0, r1, r2,
                 sem):
    wid = lax.axis_index("s") * _NC + lax.axis_index("c")
    base = wid * _BPW
    pltpu.sync_copy(i0.at[pl.ds(base, _BPW)], idx0)
    pltpu.sync_copy(i1.at[pl.ds(base, _BPW)], idx1)
    pltpu.sync_copy(i2.at[pl.ds(base, _BPW)], idx2)
    c0 = pltpu.async_copy(u0.at[idx0], r0, sem)
    c1 = pltpu.async_copy(u1.at[idx1], r1, sem)
    c2 = pltpu.async_copy(u2.at[idx2], r2, sem)
    c0.wait()
    c1.wait()
    c2.wait()
    pltpu.sync_copy(r0, out.at[0, pl.ds(base, _BPW)])
    pltpu.sync_copy(r1, out.at[1, pl.ds(base, _BPW)])
    pltpu.sync_copy(r2, out.at[2, pl.ds(base, _BPW)])


@functools.cache
def _sc_gather():
    return pl.kernel(
        _gather_body,
        out_type=jax.ShapeDtypeStruct((NMOD, B, R), jnp.float32),
        mesh=plsc.VectorSubcoreMesh(core_axis_name="c", subcore_axis_name="s",
                                    num_cores=_NC, num_subcores=_NS),
        scratch_types=[
            pltpu.VMEM((_BPW,), jnp.int32),
            pltpu.VMEM((_BPW,), jnp.int32),
            pltpu.VMEM((_BPW,), jnp.int32),
            pltpu.VMEM((_BPW, R), jnp.float32),
            pltpu.VMEM((_BPW, R), jnp.float32),
            pltpu.VMEM((_BPW, R), jnp.float32),
            pltpu.SemaphoreType.DMA,
        ],
        compiler_params=pltpu.CompilerParams(use_tc_tiling_on_sc=False),
    )


def _mlp_body(g_ref, t_ref, wff_ref, wout_ref, y_ref):
    w = wff_ref[...]
    dot = functools.partial(jnp.dot, preferred_element_type=jnp.float32)
    proj = (
        dot(g_ref[0], w[0:R])
        + dot(g_ref[1], w[R:2 * R])
        + dot(g_ref[2], w[2 * R:3 * R])
        + dot(t_ref[...], w[3 * R:3 * R + 1])
    )
    scale = 1.0 / math.sqrt(NFF)
    wo = wout_ref[...]
    y = dot(jnp.sin(proj), wo[0:NFF]) + dot(jnp.cos(proj), wo[NFF:2 * NFF])
    y_ref[...] = y * scale


_mlp = pl.pallas_call(
    _mlp_body,
    out_shape=jax.ShapeDtypeStruct((B, 1), jnp.float32),
)


def kernel(b_i_n, b_t_n, U0, U1, U2, W_ff, w_out):
    idx = b_i_n.astype(jnp.int32)
    g = _sc_gather()(idx[:, 0], idx[:, 1], idx[:, 2], U0, U1, U2)
    return _mlp(g, b_t_n.reshape(B, 1), W_ff, w_out)
